# baseline (speedup 1.0000x reference)
import functools

import jax
import jax.numpy as jnp
from jax import lax
from jax.experimental import pallas as pl
from jax.experimental.pallas import tpu as pltpu
from jax.experimental.pallas import tpu_sc as plsc

F32 = jnp.float32
NV = 100000
NF = 200000
KS = 9
FP = 204800
WIN = 128
NCORE, NSUB = 2, 16
NW = NCORE * NSUB
NVP = 100352
RPT = NVP // NSUB
ZR = RPT // 4
EPS = 1e-5
HI = lax.Precision.HIGHEST


def _mm_body(nparts, ncorn, *refs):
    xs = refs[:nparts]
    ws = refs[nparts:2 * nparts]
    outs = refs[2 * nparts:]
    for c in range(ncorn):
        acc = None
        for p in range(nparts):
            r = jnp.dot(xs[p][...], ws[p][c], preferred_element_type=F32,
                        precision=HI)
            acc = r if acc is None else acc + r
        outs[c][...] = acc


def _mm(parts, wstk, ncorn, bn=1000):
    n = parts[0].shape[0]
    cout = wstk[0].shape[2]
    in_specs = [pl.BlockSpec((bn, p.shape[1]), lambda i: (i, 0)) for p in parts]
    in_specs += [pl.BlockSpec(w.shape, lambda i: (0, 0, 0)) for w in wstk]
    f = pl.pallas_call(
        functools.partial(_mm_body, len(parts), ncorn),
        grid=(n // bn,),
        in_specs=in_specs,
        out_specs=[pl.BlockSpec((bn, cout), lambda i: (i, 0))] * ncorn,
        out_shape=[jax.ShapeDtypeStruct((n, cout), F32)] * ncorn,
    )
    return f(*parts, *wstk)


def _stats_body(nparts, has_nf, *refs):
    x = refs[0][...]
    for i in range(1, nparts):
        x = x + refs[i][...]
    if has_nf:
        x = x / refs[nparts][...]
    o_ref = refs[-1]
    s = jnp.sum(x, axis=0, keepdims=True)
    q = jnp.sum(x * x, axis=0, keepdims=True)
    upd = jnp.concatenate([s, q, jnp.zeros((6, x.shape[1]), F32)], axis=0)

    @pl.when(pl.program_id(0) == 0)
    def _():
        o_ref[...] = jnp.zeros_like(o_ref)

    o_ref[...] += upd


def _stats(ys, rows, nf=None, bn=1000):
    c = ys[0].shape[1]
    in_specs = [pl.BlockSpec((bn, c), lambda i: (i, 0)) for _ in ys]
    args = list(ys)
    if nf is not None:
        in_specs.append(pl.BlockSpec((bn, 1), lambda i: (i, 0)))
        args.append(nf)
    f = pl.pallas_call(
        functools.partial(_stats_body, len(ys), nf is not None),
        grid=(rows // bn,),
        in_specs=in_specs,
        out_specs=pl.BlockSpec((8, c), lambda i: (0, 0)),
        out_shape=jax.ShapeDtypeStruct((8, c), F32),
    )
    return f(*args)


def _bn_ss(st, rows, g, be):
    mu = st[0] / rows
    var = st[1] / rows - mu * mu
    sc = g * lax.rsqrt(var + EPS)
    sh = be - mu * sc
    return jnp.concatenate([sc[None], sh[None], jnp.zeros((6, g.shape[0]), F32)],
                           axis=0)


def _ftf_body(ngrp, y_ref, cf_ref, ss_ref, kw_ref, pw_ref, *o_refs):
    y = jnp.maximum(y_ref[...] * ss_ref[0:1, :] + ss_ref[1:2, :], 0.0)
    s = jnp.dot(cf_ref[...], kw_ref[...], preferred_element_type=F32,
                precision=HI)
    w = jnp.dot(y * s, pw_ref[...], preferred_element_type=F32, precision=HI)
    for g in range(ngrp):
        o_refs[g][...] = w[:, g * 16:(g + 1) * 16]


def _ftf(ypre, cfp, ss, kwp, pw2, bn=1024):
    cout = pw2.shape[1]
    ngrp = cout // 16
    f = pl.pallas_call(
        functools.partial(_ftf_body, ngrp),
        grid=(FP // bn,),
        in_specs=[pl.BlockSpec((bn, 128), lambda i: (i, 0)),
                  pl.BlockSpec((bn, 128), lambda i: (i, 0)),
                  pl.BlockSpec((8, 128), lambda i: (0, 0)),
                  pl.BlockSpec((128, 128), lambda i: (0, 0)),
                  pl.BlockSpec((128, cout), lambda i: (0, 0))],
        out_specs=[pl.BlockSpec((bn, 16), lambda i: (i, 0))] * ngrp,
        out_shape=[jax.ShapeDtypeStruct((FP, 16), F32)] * ngrp,
    )
    return f(ypre, cfp, ss, kwp, pw2)


def _norm_body(nparts, has_nf, *refs):
    x = refs[0][...]
    for i in range(1, nparts):
        x = x + refs[i][...]
    if has_nf:
        x = x / refs[nparts][...]
    ss_ref, o_ref = refs[-2], refs[-1]
    o_ref[...] = jnp.maximum(x * ss_ref[0:1, :] + ss_ref[1:2, :], 0.0)


def _norm(xs, ss, nf=None, bn=1000):
    c = xs[0].shape[1]
    in_specs = [pl.BlockSpec((bn, c), lambda i: (i, 0)) for _ in xs]
    args = list(xs)
    if nf is not None:
        in_specs.append(pl.BlockSpec((bn, 1), lambda i: (i, 0)))
        args.append(nf)
    in_specs.append(pl.BlockSpec((8, c), lambda i: (0, 0)))
    args.append(ss)
    f = pl.pallas_call(
        functools.partial(_norm_body, len(xs), nf is not None),
        grid=(NV // bn,),
        in_specs=in_specs,
        out_specs=pl.BlockSpec((bn, c), lambda i: (i, 0)),
        out_shape=jax.ShapeDtypeStruct((NV, c), F32),
    )
    return f(*args)


def _gstats_body(ngrp, *refs):
    x = jnp.concatenate([refs[i][...] for i in range(ngrp)], axis=1)
    x = x / refs[ngrp][...]
    o_ref = refs[-1]
    s = jnp.sum(x, axis=0, keepdims=True)
    q = jnp.sum(x * x, axis=0, keepdims=True)
    upd = jnp.concatenate([s, q, jnp.zeros((6, x.shape[1]), F32)], axis=0)

    @pl.when(pl.program_id(0) == 0)
    def _():
        o_ref[...] = jnp.zeros_like(o_ref)

    o_ref[...] += upd


def _gstats(groups, nf, bn=1000):
    ngrp = len(groups)
    c = 16 * ngrp
    in_specs = [pl.BlockSpec((bn, 16), lambda i: (i, 0)) for _ in groups]
    in_specs.append(pl.BlockSpec((bn, 1), lambda i: (i, 0)))
    f = pl.pallas_call(
        functools.partial(_gstats_body, ngrp),
        grid=(NV // bn,),
        in_specs=in_specs,
        out_specs=pl.BlockSpec((8, c), lambda i: (0, 0)),
        out_shape=jax.ShapeDtypeStruct((8, c), F32),
    )
    return f(*groups, nf)


def _gnorm_body(ngrp, *refs):
    x = jnp.concatenate([refs[i][...] for i in range(ngrp)], axis=1)
    x = x / refs[ngrp][...]
    ss_ref, o_ref = refs[-2], refs[-1]
    o_ref[...] = jnp.maximum(x * ss_ref[0:1, :] + ss_ref[1:2, :], 0.0)


def _gnorm(groups, ss, nf, bn=1000):
    ngrp = len(groups)
    c = 16 * ngrp
    in_specs = [pl.BlockSpec((bn, 16), lambda i: (i, 0)) for _ in groups]
    in_specs.append(pl.BlockSpec((bn, 1), lambda i: (i, 0)))
    in_specs.append(pl.BlockSpec((8, c), lambda i: (0, 0)))
    f = pl.pallas_call(
        functools.partial(_gnorm_body, ngrp),
        grid=(NV // bn,),
        in_specs=in_specs,
        out_specs=pl.BlockSpec((bn, c), lambda i: (i, 0)),
        out_shape=jax.ShapeDtypeStruct((NV, c), F32),
    )
    return f(*groups, nf, ss)


def _sc_gather(u0, u1, u2, ft3):
    per = (FP // WIN) // NW

    @functools.partial(
        pl.kernel,
        out_type=jax.ShapeDtypeStruct((FP, 128), F32),
        mesh=plsc.VectorSubcoreMesh(core_axis_name="c", subcore_axis_name="s"),
        scratch_types=[pltpu.VMEM((3, 1, WIN), jnp.int32),
                       pltpu.VMEM((WIN, 128), F32),
                       pltpu.VMEM((WIN, 128), F32),
                       pltpu.VMEM((WIN, 128), F32),
                       pltpu.SemaphoreType.DMA,
                       pltpu.SemaphoreType.DMA,
                       pltpu.SemaphoreType.DMA])
    def k(u0_h, u1_h, u2_h, ft_h, o_h, idx_v, b0, b1, b2, s0, s1, s2):
        wid = lax.axis_index("s") * NCORE + lax.axis_index("c")

        @pl.loop(0, per)
        def _(j):
            base = (wid * per + j) * WIN
            pltpu.sync_copy(ft_h.at[:, :, pl.ds(base, WIN)], idx_v)
            c0 = pltpu.async_copy(u0_h.at[idx_v.at[0, 0]], b0, s0)
            c0.wait()
            pltpu.sync_copy(u1_h.at[idx_v.at[1, 0]], b0, add=True)
            pltpu.sync_copy(u2_h.at[idx_v.at[2, 0]], b0, add=True)
            pltpu.sync_copy(b0, o_h.at[pl.ds(base, WIN)])

    return k(u0, u1, u2, ft3)


def _sc_scatter(wgs, ft3):
    ngrp = len(wgs)
    gpc = ngrp // NCORE
    nps = (FP // WIN) // NSUB
    zb = RPT // 4

    @functools.partial(
        pl.kernel,
        out_type=[jax.ShapeDtypeStruct((NVP, 16), F32)] * ngrp,
        mesh=plsc.VectorSubcoreMesh(core_axis_name="c", subcore_axis_name="s"),
        compiler_params=pltpu.CompilerParams(use_tc_tiling_on_sc=False),
        scratch_types=[pltpu.VMEM((3, 1, WIN), jnp.int32),
                       pltpu.VMEM((WIN, 16), F32),
                       pltpu.VMEM((zb, 16), F32),
                       pltpu.VMEM_SHARED((NVP, 16), F32)])
    def k(*refs):
        wg_hs = refs[:ngrp]
        ft_h = refs[ngrp]
        o_hs = refs[ngrp + 1:ngrp + 1 + ngrp]
        idx_v, wbuf, zbuf, shared = refs[ngrp + 1 + ngrp:]
        cid = lax.axis_index("c")
        sid = lax.axis_index("s")

        @pl.loop(0, zb)
        def _(r):
            zbuf[r, pl.ds(0, 16)] = jnp.zeros((16,), F32)

        rbase = sid * RPT

        def do_group(g):
            for zz in range(4):
                pltpu.sync_copy(zbuf, shared.at[pl.ds(rbase + zz * zb, zb)])
            plsc.subcore_barrier()

            @pl.loop(0, nps)
            def _(j):
                win = sid * nps + j
                pltpu.sync_copy(ft_h.at[:, :, pl.ds(win * WIN, WIN)], idx_v)
                pltpu.sync_copy(wg_hs[g].at[pl.ds(win * WIN, WIN)], wbuf)
                for c in range(3):
                    pltpu.sync_copy(wbuf, shared.at[idx_v.at[c, 0]], add=True)

            plsc.subcore_barrier()
            pltpu.sync_copy(shared.at[pl.ds(rbase, RPT)],
                            o_hs[g].at[pl.ds(rbase, RPT)])

        for ci in range(NCORE):
            @pl.when(cid == ci)
            def _(ci=ci):
                for g in range(ci * gpc, (ci + 1) * gpc):
                    do_group(g)

    return k(*wgs, ft3)


def _layer(parts, wstk, kwp, pw2, gf, bef, gv, bev, ft3, cfp, nf2):
    cout = pw2.shape[1]
    ngrp = cout // 16
    u0, u1, u2 = _mm(parts, wstk, 3)
    ypre = _sc_gather(u0, u1, u2, ft3)
    ssf = _bn_ss(_stats([ypre], NF), NF, gf, bef)
    wgs = _ftf(ypre, cfp, ssf, kwp, pw2)
    aggs = _sc_scatter(wgs, ft3)
    ssv = _bn_ss(_gstats(aggs, nf2), NV, gv, bev)
    return _gnorm(aggs, ssv, nf2)


def kernel(inputs, face, full_nf_count, full_vt_map, filt_coeff, params):
    p = params
    face32 = face.astype(jnp.int32)
    ft3 = jnp.pad(face32.T, ((0, 0), (0, FP - NF))).reshape(3, 1, FP)
    cfp = jnp.pad(filt_coeff, ((0, FP - NF), (0, 128 - KS)))
    nf2 = full_nf_count.reshape(NV, 1)

    def wstack(i, cin_parts):
        dw = p['c%d_dw' % i]
        pw = p['c%d_v2f_pw' % i]
        full = dw[:, :, None] * pw[None, :, :]
        outs = []
        ofs = 0
        for cp in cin_parts:
            outs.append(full[:, ofs:ofs + cp, :])
            ofs += cp
        return outs

    def kwpad(i):
        return jnp.pad(p['c%d_kw' % i], ((0, 128 - KS), (0, 0)))

    def largs(i):
        return (kwpad(i), p['c%d_f2v_pw' % i],
                p['c%d_v2f_g' % i], p['c%d_v2f_be' % i],
                p['c%d_f2v_g' % i], p['c%d_f2v_be' % i],
                ft3, cfp, nf2)

    x0 = inputs
    neta = _layer([x0], wstack(0, [128]), *largs(0))
    netb = _layer([neta], wstack(1, [128]), *largs(1))
    netc = _layer([x0, netb], wstack(2, [128, 32]), *largs(2))
    netd = _layer([netc], wstack(3, [128]), *largs(3))

    tw = p['t_w']
    t = _mm([x0, netb, netd],
            [tw[None, :128], tw[None, 128:160], tw[None, 160:]], 1)[0]
    sst = _bn_ss(_stats([t], NV), NV, p['t_g'], p['t_be'])
    return _norm([t], sst)


# speedup vs baseline: 1.1417x; 1.1417x over previous
import functools

import jax
import jax.numpy as jnp
from jax import lax
from jax.experimental import pallas as pl
from jax.experimental.pallas import tpu as pltpu
from jax.experimental.pallas import tpu_sc as plsc

F32 = jnp.float32
NV = 100000
NF = 200000
KS = 9
FP = 204800
WIN = 128
NCORE, NSUB = 2, 16
NW = NCORE * NSUB
NVP = 100352
RPT = NVP // NSUB
ZR = RPT // 4
EPS = 1e-5
HI = lax.Precision.HIGHEST


def _mm_body(nparts, ncorn, *refs):
    xs = refs[:nparts]
    ws = refs[nparts:2 * nparts]
    outs = refs[2 * nparts:]
    for c in range(ncorn):
        acc = None
        for p in range(nparts):
            r = jnp.dot(xs[p][...], ws[p][c], preferred_element_type=F32,
                        precision=HI)
            acc = r if acc is None else acc + r
        outs[c][...] = acc


def _mm(parts, wstk, ncorn, bn=1000):
    n = parts[0].shape[0]
    cout = wstk[0].shape[2]
    in_specs = [pl.BlockSpec((bn, p.shape[1]), lambda i: (i, 0)) for p in parts]
    in_specs += [pl.BlockSpec(w.shape, lambda i: (0, 0, 0)) for w in wstk]
    f = pl.pallas_call(
        functools.partial(_mm_body, len(parts), ncorn),
        grid=(n // bn,),
        in_specs=in_specs,
        out_specs=[pl.BlockSpec((bn, cout), lambda i: (i, 0))] * ncorn,
        out_shape=[jax.ShapeDtypeStruct((n, cout), F32)] * ncorn,
    )
    return f(*parts, *wstk)


def _stats_body(nparts, has_nf, *refs):
    x = refs[0][...]
    for i in range(1, nparts):
        x = x + refs[i][...]
    if has_nf:
        x = x / refs[nparts][...]
    o_ref = refs[-1]
    s = jnp.sum(x, axis=0, keepdims=True)
    q = jnp.sum(x * x, axis=0, keepdims=True)
    upd = jnp.concatenate([s, q, jnp.zeros((6, x.shape[1]), F32)], axis=0)

    @pl.when(pl.program_id(0) == 0)
    def _():
        o_ref[...] = jnp.zeros_like(o_ref)

    o_ref[...] += upd


def _stats(ys, rows, nf=None, bn=1000):
    c = ys[0].shape[1]
    in_specs = [pl.BlockSpec((bn, c), lambda i: (i, 0)) for _ in ys]
    args = list(ys)
    if nf is not None:
        in_specs.append(pl.BlockSpec((bn, 1), lambda i: (i, 0)))
        args.append(nf)
    f = pl.pallas_call(
        functools.partial(_stats_body, len(ys), nf is not None),
        grid=(rows // bn,),
        in_specs=in_specs,
        out_specs=pl.BlockSpec((8, c), lambda i: (0, 0)),
        out_shape=jax.ShapeDtypeStruct((8, c), F32),
    )
    return f(*args)


def _bn_ss(st, rows, g, be):
    mu = st[0] / rows
    var = st[1] / rows - mu * mu
    sc = g * lax.rsqrt(var + EPS)
    sh = be - mu * sc
    return jnp.concatenate([sc[None], sh[None], jnp.zeros((6, g.shape[0]), F32)],
                           axis=0)


def _ftf_body(ngrp, y_ref, cf_ref, ss_ref, kw_ref, pw_ref, *o_refs):
    y = jnp.maximum(y_ref[...] * ss_ref[0:1, :] + ss_ref[1:2, :], 0.0)
    s = jnp.dot(cf_ref[...], kw_ref[...], preferred_element_type=F32,
                precision=HI)
    w = jnp.dot(y * s, pw_ref[...], preferred_element_type=F32, precision=HI)
    for g in range(ngrp):
        o_refs[g][...] = w[:, g * 16:(g + 1) * 16]


def _ftf(ypre, cfp, ss, kwp, pw2, bn=1024):
    cout = pw2.shape[1]
    ngrp = cout // 16
    f = pl.pallas_call(
        functools.partial(_ftf_body, ngrp),
        grid=(FP // bn,),
        in_specs=[pl.BlockSpec((bn, 128), lambda i: (i, 0)),
                  pl.BlockSpec((bn, 128), lambda i: (i, 0)),
                  pl.BlockSpec((8, 128), lambda i: (0, 0)),
                  pl.BlockSpec((128, 128), lambda i: (0, 0)),
                  pl.BlockSpec((128, cout), lambda i: (0, 0))],
        out_specs=[pl.BlockSpec((bn, 16), lambda i: (i, 0))] * ngrp,
        out_shape=[jax.ShapeDtypeStruct((FP, 16), F32)] * ngrp,
    )
    return f(ypre, cfp, ss, kwp, pw2)


def _norm_body(nparts, has_nf, *refs):
    x = refs[0][...]
    for i in range(1, nparts):
        x = x + refs[i][...]
    if has_nf:
        x = x / refs[nparts][...]
    ss_ref, o_ref = refs[-2], refs[-1]
    o_ref[...] = jnp.maximum(x * ss_ref[0:1, :] + ss_ref[1:2, :], 0.0)


def _norm(xs, ss, nf=None, bn=1000):
    c = xs[0].shape[1]
    in_specs = [pl.BlockSpec((bn, c), lambda i: (i, 0)) for _ in xs]
    args = list(xs)
    if nf is not None:
        in_specs.append(pl.BlockSpec((bn, 1), lambda i: (i, 0)))
        args.append(nf)
    in_specs.append(pl.BlockSpec((8, c), lambda i: (0, 0)))
    args.append(ss)
    f = pl.pallas_call(
        functools.partial(_norm_body, len(xs), nf is not None),
        grid=(NV // bn,),
        in_specs=in_specs,
        out_specs=pl.BlockSpec((bn, c), lambda i: (i, 0)),
        out_shape=jax.ShapeDtypeStruct((NV, c), F32),
    )
    return f(*args)


def _gstats_body(ngrp, *refs):
    x = jnp.concatenate([refs[i][...] for i in range(ngrp)], axis=1)
    x = x / refs[ngrp][...]
    o_ref = refs[-1]
    s = jnp.sum(x, axis=0, keepdims=True)
    q = jnp.sum(x * x, axis=0, keepdims=True)
    upd = jnp.concatenate([s, q, jnp.zeros((6, x.shape[1]), F32)], axis=0)

    @pl.when(pl.program_id(0) == 0)
    def _():
        o_ref[...] = jnp.zeros_like(o_ref)

    o_ref[...] += upd


def _gstats(groups, nf, bn=1000):
    ngrp = len(groups)
    c = 16 * ngrp
    in_specs = [pl.BlockSpec((bn, 16), lambda i: (i, 0)) for _ in groups]
    in_specs.append(pl.BlockSpec((bn, 1), lambda i: (i, 0)))
    f = pl.pallas_call(
        functools.partial(_gstats_body, ngrp),
        grid=(NV // bn,),
        in_specs=in_specs,
        out_specs=pl.BlockSpec((8, c), lambda i: (0, 0)),
        out_shape=jax.ShapeDtypeStruct((8, c), F32),
    )
    return f(*groups, nf)


def _gnorm_body(ngrp, *refs):
    x = jnp.concatenate([refs[i][...] for i in range(ngrp)], axis=1)
    x = x / refs[ngrp][...]
    ss_ref, o_ref = refs[-2], refs[-1]
    o_ref[...] = jnp.maximum(x * ss_ref[0:1, :] + ss_ref[1:2, :], 0.0)


def _gnorm(groups, ss, nf, bn=1000):
    ngrp = len(groups)
    c = 16 * ngrp
    in_specs = [pl.BlockSpec((bn, 16), lambda i: (i, 0)) for _ in groups]
    in_specs.append(pl.BlockSpec((bn, 1), lambda i: (i, 0)))
    in_specs.append(pl.BlockSpec((8, c), lambda i: (0, 0)))
    f = pl.pallas_call(
        functools.partial(_gnorm_body, ngrp),
        grid=(NV // bn,),
        in_specs=in_specs,
        out_specs=pl.BlockSpec((bn, c), lambda i: (i, 0)),
        out_shape=jax.ShapeDtypeStruct((NV, c), F32),
    )
    return f(*groups, nf, ss)


def _sc_gather(u0, u1, u2, ft3):
    per = (FP // WIN) // NW

    @functools.partial(
        pl.kernel,
        out_type=jax.ShapeDtypeStruct((FP, 128), F32),
        mesh=plsc.VectorSubcoreMesh(core_axis_name="c", subcore_axis_name="s"),
        scratch_types=[pltpu.VMEM((3, 1, WIN), jnp.int32),
                       pltpu.VMEM((WIN, 128), F32),
                       pltpu.VMEM((WIN, 128), F32),
                       pltpu.VMEM((WIN, 128), F32),
                       pltpu.SemaphoreType.DMA,
                       pltpu.SemaphoreType.DMA,
                       pltpu.SemaphoreType.DMA])
    def k(u0_h, u1_h, u2_h, ft_h, o_h, idx_v, b0, b1, b2, s0, s1, s2):
        wid = lax.axis_index("s") * NCORE + lax.axis_index("c")

        @pl.loop(0, per)
        def _(j):
            base = (wid * per + j) * WIN
            pltpu.sync_copy(ft_h.at[:, :, pl.ds(base, WIN)], idx_v)
            c0 = pltpu.async_copy(u0_h.at[idx_v.at[0, 0]], b0, s0)
            c1 = pltpu.async_copy(u1_h.at[idx_v.at[1, 0]], b1, s1)
            c2 = pltpu.async_copy(u2_h.at[idx_v.at[2, 0]], b2, s2)
            c0.wait()
            c1.wait()
            c2.wait()

            @pl.loop(0, WIN)
            def _(r):
                for kk in range(8):
                    sl = pl.ds(kk * 16, 16)
                    b0[r, sl] = b0[r, sl] + b1[r, sl] + b2[r, sl]

            pltpu.sync_copy(b0, o_h.at[pl.ds(base, WIN)])

    return k(u0, u1, u2, ft3)


def _sc_scatter(wgs, ft3):
    ngrp = len(wgs)
    gpc = ngrp // NCORE
    nps = (FP // WIN) // NSUB
    zb = RPT // 4

    @functools.partial(
        pl.kernel,
        out_type=[jax.ShapeDtypeStruct((NVP, 16), F32)] * ngrp,
        mesh=plsc.VectorSubcoreMesh(core_axis_name="c", subcore_axis_name="s"),
        compiler_params=pltpu.CompilerParams(use_tc_tiling_on_sc=False),
        scratch_types=[pltpu.VMEM((3, 1, WIN), jnp.int32),
                       pltpu.VMEM((WIN, 16), F32),
                       pltpu.VMEM((zb, 16), F32),
                       pltpu.VMEM_SHARED((NVP, 16), F32)])
    def k(*refs):
        wg_hs = refs[:ngrp]
        ft_h = refs[ngrp]
        o_hs = refs[ngrp + 1:ngrp + 1 + ngrp]
        idx_v, wbuf, zbuf, shared = refs[ngrp + 1 + ngrp:]
        cid = lax.axis_index("c")
        sid = lax.axis_index("s")

        @pl.loop(0, zb)
        def _(r):
            zbuf[r, pl.ds(0, 16)] = jnp.zeros((16,), F32)

        rbase = sid * RPT

        def do_group(g):
            for zz in range(4):
                pltpu.sync_copy(zbuf, shared.at[pl.ds(rbase + zz * zb, zb)])
            plsc.subcore_barrier()

            @pl.loop(0, nps)
            def _(j):
                win = sid * nps + j
                pltpu.sync_copy(ft_h.at[:, :, pl.ds(win * WIN, WIN)], idx_v)
                pltpu.sync_copy(wg_hs[g].at[pl.ds(win * WIN, WIN)], wbuf)
                for c in range(3):
                    pltpu.sync_copy(wbuf, shared.at[idx_v.at[c, 0]], add=True)

            plsc.subcore_barrier()
            pltpu.sync_copy(shared.at[pl.ds(rbase, RPT)],
                            o_hs[g].at[pl.ds(rbase, RPT)])

        for ci in range(NCORE):
            @pl.when(cid == ci)
            def _(ci=ci):
                for g in range(ci * gpc, (ci + 1) * gpc):
                    do_group(g)

    return k(*wgs, ft3)


def _layer(parts, wstk, kwp, pw2, gf, bef, gv, bev, ft3, cfp, nf2):
    cout = pw2.shape[1]
    ngrp = cout // 16
    u0, u1, u2 = _mm(parts, wstk, 3)
    ypre = _sc_gather(u0, u1, u2, ft3)
    ssf = _bn_ss(_stats([ypre], NF), NF, gf, bef)
    wgs = _ftf(ypre, cfp, ssf, kwp, pw2)
    aggs = _sc_scatter(wgs, ft3)
    ssv = _bn_ss(_gstats(aggs, nf2), NV, gv, bev)
    return _gnorm(aggs, ssv, nf2)


def kernel(inputs, face, full_nf_count, full_vt_map, filt_coeff, params):
    p = params
    face32 = face.astype(jnp.int32)
    ft3 = jnp.pad(face32.T, ((0, 0), (0, FP - NF))).reshape(3, 1, FP)
    cfp = jnp.pad(filt_coeff, ((0, FP - NF), (0, 128 - KS)))
    nf2 = full_nf_count.reshape(NV, 1)

    def wstack(i, cin_parts):
        dw = p['c%d_dw' % i]
        pw = p['c%d_v2f_pw' % i]
        full = dw[:, :, None] * pw[None, :, :]
        outs = []
        ofs = 0
        for cp in cin_parts:
            outs.append(full[:, ofs:ofs + cp, :])
            ofs += cp
        return outs

    def kwpad(i):
        return jnp.pad(p['c%d_kw' % i], ((0, 128 - KS), (0, 0)))

    def largs(i):
        return (kwpad(i), p['c%d_f2v_pw' % i],
                p['c%d_v2f_g' % i], p['c%d_v2f_be' % i],
                p['c%d_f2v_g' % i], p['c%d_f2v_be' % i],
                ft3, cfp, nf2)

    x0 = inputs
    neta = _layer([x0], wstack(0, [128]), *largs(0))
    netb = _layer([neta], wstack(1, [128]), *largs(1))
    netc = _layer([x0, netb], wstack(2, [128, 32]), *largs(2))
    netd = _layer([netc], wstack(3, [128]), *largs(3))

    tw = p['t_w']
    t = _mm([x0, netb, netd],
            [tw[None, :128], tw[None, 128:160], tw[None, 160:]], 1)[0]
    sst = _bn_ss(_stats([t], NV), NV, p['t_g'], p['t_be'])
    return _norm([t], sst)


# speedup vs baseline: 1.1431x; 1.0012x over previous
import functools

import jax
import jax.numpy as jnp
from jax import lax
from jax.experimental import pallas as pl
from jax.experimental.pallas import tpu as pltpu
from jax.experimental.pallas import tpu_sc as plsc

F32 = jnp.float32
NV = 100000
NF = 200000
KS = 9
FP = 204800
WIN = 128
NCORE, NSUB = 2, 16
NW = NCORE * NSUB
NVP = 100352
RPT = NVP // NSUB
ZR = RPT // 4
EPS = 1e-5
HI = lax.Precision.HIGHEST


def _mm_body(nparts, ncorn, *refs):
    xs = refs[:nparts]
    ws = refs[nparts:2 * nparts]
    outs = refs[2 * nparts:]
    for c in range(ncorn):
        acc = None
        for p in range(nparts):
            r = jnp.dot(xs[p][...], ws[p][c], preferred_element_type=F32,
                        precision=HI)
            acc = r if acc is None else acc + r
        outs[c][...] = acc


def _mm(parts, wstk, ncorn, bn=1000):
    n = parts[0].shape[0]
    cout = wstk[0].shape[2]
    in_specs = [pl.BlockSpec((bn, p.shape[1]), lambda i: (i, 0)) for p in parts]
    in_specs += [pl.BlockSpec(w.shape, lambda i: (0, 0, 0)) for w in wstk]
    f = pl.pallas_call(
        functools.partial(_mm_body, len(parts), ncorn),
        grid=(n // bn,),
        in_specs=in_specs,
        out_specs=[pl.BlockSpec((bn, cout), lambda i: (i, 0))] * ncorn,
        out_shape=[jax.ShapeDtypeStruct((n, cout), F32)] * ncorn,
    )
    return f(*parts, *wstk)


def _stats_body(nparts, has_nf, *refs):
    x = refs[0][...]
    for i in range(1, nparts):
        x = x + refs[i][...]
    if has_nf:
        x = x / refs[nparts][...]
    o_ref = refs[-1]
    s = jnp.sum(x, axis=0, keepdims=True)
    q = jnp.sum(x * x, axis=0, keepdims=True)
    upd = jnp.concatenate([s, q, jnp.zeros((6, x.shape[1]), F32)], axis=0)

    @pl.when(pl.program_id(0) == 0)
    def _():
        o_ref[...] = jnp.zeros_like(o_ref)

    o_ref[...] += upd


def _stats(ys, rows, nf=None, bn=1000):
    c = ys[0].shape[1]
    in_specs = [pl.BlockSpec((bn, c), lambda i: (i, 0)) for _ in ys]
    args = list(ys)
    if nf is not None:
        in_specs.append(pl.BlockSpec((bn, 1), lambda i: (i, 0)))
        args.append(nf)
    f = pl.pallas_call(
        functools.partial(_stats_body, len(ys), nf is not None),
        grid=(rows // bn,),
        in_specs=in_specs,
        out_specs=pl.BlockSpec((8, c), lambda i: (0, 0)),
        out_shape=jax.ShapeDtypeStruct((8, c), F32),
    )
    return f(*args)


def _bn_ss(st, rows, g, be):
    mu = st[0] / rows
    var = st[1] / rows - mu * mu
    sc = g * lax.rsqrt(var + EPS)
    sh = be - mu * sc
    return jnp.concatenate([sc[None], sh[None], jnp.zeros((6, g.shape[0]), F32)],
                           axis=0)


def _ftf_body(ngrp, gw, y_ref, cf_ref, ss_ref, kw_ref, pw_ref, *o_refs):
    y = jnp.maximum(y_ref[...] * ss_ref[0:1, :] + ss_ref[1:2, :], 0.0)
    s = jnp.dot(cf_ref[...], kw_ref[...], preferred_element_type=F32,
                precision=HI)
    w = jnp.dot(y * s, pw_ref[...], preferred_element_type=F32, precision=HI)
    for g in range(ngrp):
        o_refs[g][...] = w[:, g * gw:(g + 1) * gw]


def _ftf(ypre, cfp, ss, kwp, pw2, gw, bn=1024):
    cout = pw2.shape[1]
    ngrp = cout // gw
    f = pl.pallas_call(
        functools.partial(_ftf_body, ngrp, gw),
        grid=(FP // bn,),
        in_specs=[pl.BlockSpec((bn, 128), lambda i: (i, 0)),
                  pl.BlockSpec((bn, 128), lambda i: (i, 0)),
                  pl.BlockSpec((8, 128), lambda i: (0, 0)),
                  pl.BlockSpec((128, 128), lambda i: (0, 0)),
                  pl.BlockSpec((128, cout), lambda i: (0, 0))],
        out_specs=[pl.BlockSpec((bn, gw), lambda i: (i, 0))] * ngrp,
        out_shape=[jax.ShapeDtypeStruct((FP, gw), F32)] * ngrp,
    )
    return f(ypre, cfp, ss, kwp, pw2)


def _norm_body(nparts, has_nf, *refs):
    x = refs[0][...]
    for i in range(1, nparts):
        x = x + refs[i][...]
    if has_nf:
        x = x / refs[nparts][...]
    ss_ref, o_ref = refs[-2], refs[-1]
    o_ref[...] = jnp.maximum(x * ss_ref[0:1, :] + ss_ref[1:2, :], 0.0)


def _norm(xs, ss, nf=None, bn=1000):
    c = xs[0].shape[1]
    in_specs = [pl.BlockSpec((bn, c), lambda i: (i, 0)) for _ in xs]
    args = list(xs)
    if nf is not None:
        in_specs.append(pl.BlockSpec((bn, 1), lambda i: (i, 0)))
        args.append(nf)
    in_specs.append(pl.BlockSpec((8, c), lambda i: (0, 0)))
    args.append(ss)
    f = pl.pallas_call(
        functools.partial(_norm_body, len(xs), nf is not None),
        grid=(NV // bn,),
        in_specs=in_specs,
        out_specs=pl.BlockSpec((bn, c), lambda i: (i, 0)),
        out_shape=jax.ShapeDtypeStruct((NV, c), F32),
    )
    return f(*args)


def _gstats_body(ngrp, *refs):
    x = jnp.concatenate([refs[i][...] for i in range(ngrp)], axis=1)
    x = x / refs[ngrp][...]
    o_ref = refs[-1]
    s = jnp.sum(x, axis=0, keepdims=True)
    q = jnp.sum(x * x, axis=0, keepdims=True)
    upd = jnp.concatenate([s, q, jnp.zeros((6, x.shape[1]), F32)], axis=0)

    @pl.when(pl.program_id(0) == 0)
    def _():
        o_ref[...] = jnp.zeros_like(o_ref)

    o_ref[...] += upd


def _gstats(groups, nf, gw, bn=1000):
    ngrp = len(groups)
    c = gw * ngrp
    in_specs = [pl.BlockSpec((bn, gw), lambda i: (i, 0)) for _ in groups]
    in_specs.append(pl.BlockSpec((bn, 1), lambda i: (i, 0)))
    f = pl.pallas_call(
        functools.partial(_gstats_body, ngrp),
        grid=(NV // bn,),
        in_specs=in_specs,
        out_specs=pl.BlockSpec((8, c), lambda i: (0, 0)),
        out_shape=jax.ShapeDtypeStruct((8, c), F32),
    )
    return f(*groups, nf)


def _gnorm_body(ngrp, *refs):
    x = jnp.concatenate([refs[i][...] for i in range(ngrp)], axis=1)
    x = x / refs[ngrp][...]
    ss_ref, o_ref = refs[-2], refs[-1]
    o_ref[...] = jnp.maximum(x * ss_ref[0:1, :] + ss_ref[1:2, :], 0.0)


def _gnorm(groups, ss, nf, gw, bn=1000):
    ngrp = len(groups)
    c = gw * ngrp
    in_specs = [pl.BlockSpec((bn, gw), lambda i: (i, 0)) for _ in groups]
    in_specs.append(pl.BlockSpec((bn, 1), lambda i: (i, 0)))
    in_specs.append(pl.BlockSpec((8, c), lambda i: (0, 0)))
    f = pl.pallas_call(
        functools.partial(_gnorm_body, ngrp),
        grid=(NV // bn,),
        in_specs=in_specs,
        out_specs=pl.BlockSpec((bn, c), lambda i: (i, 0)),
        out_shape=jax.ShapeDtypeStruct((NV, c), F32),
    )
    return f(*groups, nf, ss)


def _sc_gather(u0, u1, u2, ft3):
    per = (FP // WIN) // NW

    @functools.partial(
        pl.kernel,
        out_type=jax.ShapeDtypeStruct((FP, 128), F32),
        mesh=plsc.VectorSubcoreMesh(core_axis_name="c", subcore_axis_name="s"),
        scratch_types=[pltpu.VMEM((3, 1, WIN), jnp.int32),
                       pltpu.VMEM((WIN, 128), F32),
                       pltpu.VMEM((WIN, 128), F32),
                       pltpu.VMEM((WIN, 128), F32),
                       pltpu.SemaphoreType.DMA,
                       pltpu.SemaphoreType.DMA,
                       pltpu.SemaphoreType.DMA])
    def k(u0_h, u1_h, u2_h, ft_h, o_h, idx_v, b0, b1, b2, s0, s1, s2):
        wid = lax.axis_index("s") * NCORE + lax.axis_index("c")

        @pl.loop(0, per)
        def _(j):
            base = (wid * per + j) * WIN
            pltpu.sync_copy(ft_h.at[:, :, pl.ds(base, WIN)], idx_v)
            c0 = pltpu.async_copy(u0_h.at[idx_v.at[0, 0]], b0, s0)
            c1 = pltpu.async_copy(u1_h.at[idx_v.at[1, 0]], b1, s1)
            c2 = pltpu.async_copy(u2_h.at[idx_v.at[2, 0]], b2, s2)
            c0.wait()
            c1.wait()
            c2.wait()

            @pl.loop(0, WIN)
            def _(r):
                for kk in range(8):
                    sl = pl.ds(kk * 16, 16)
                    b0[r, sl] = b0[r, sl] + b1[r, sl] + b2[r, sl]

            pltpu.sync_copy(b0, o_h.at[pl.ds(base, WIN)])

    return k(u0, u1, u2, ft3)


def _sc_scatter(wgs, ft3, gw):
    ngrp = len(wgs)
    gpc = ngrp // NCORE
    nps = (FP // WIN) // NSUB
    zb = RPT // 4

    @functools.partial(
        pl.kernel,
        out_type=[jax.ShapeDtypeStruct((NVP, gw), F32)] * ngrp,
        mesh=plsc.VectorSubcoreMesh(core_axis_name="c", subcore_axis_name="s"),
        compiler_params=pltpu.CompilerParams(use_tc_tiling_on_sc=False),
        scratch_types=[pltpu.VMEM((3, 1, WIN), jnp.int32),
                       pltpu.VMEM((WIN, gw), F32),
                       pltpu.VMEM((zb, gw), F32),
                       pltpu.VMEM_SHARED((NVP, gw), F32)])
    def k(*refs):
        wg_hs = refs[:ngrp]
        ft_h = refs[ngrp]
        o_hs = refs[ngrp + 1:ngrp + 1 + ngrp]
        idx_v, wbuf, zbuf, shared = refs[ngrp + 1 + ngrp:]
        cid = lax.axis_index("c")
        sid = lax.axis_index("s")

        @pl.loop(0, zb)
        def _(r):
            for q in range(gw // 16):
                zbuf[r, pl.ds(q * 16, 16)] = jnp.zeros((16,), F32)

        rbase = sid * RPT

        def do_group(g):
            for zz in range(4):
                pltpu.sync_copy(zbuf, shared.at[pl.ds(rbase + zz * zb, zb)])
            plsc.subcore_barrier()

            @pl.loop(0, nps)
            def _(j):
                win = sid * nps + j
                pltpu.sync_copy(ft_h.at[:, :, pl.ds(win * WIN, WIN)], idx_v)
                pltpu.sync_copy(wg_hs[g].at[pl.ds(win * WIN, WIN)], wbuf)
                for c in range(3):
                    pltpu.sync_copy(wbuf, shared.at[idx_v.at[c, 0]], add=True)

            plsc.subcore_barrier()
            pltpu.sync_copy(shared.at[pl.ds(rbase, RPT)],
                            o_hs[g].at[pl.ds(rbase, RPT)])

        for ci in range(NCORE):
            @pl.when(cid == ci)
            def _(ci=ci):
                for g in range(ci * gpc, (ci + 1) * gpc):
                    do_group(g)

    return k(*wgs, ft3)


def _layer(parts, wstk, kwp, pw2, gf, bef, gv, bev, ft3, cfp, nf2):
    cout = pw2.shape[1]
    ngrp = cout // 16
    u0, u1, u2 = _mm(parts, wstk, 3)
    ypre = _sc_gather(u0, u1, u2, ft3)
    ssf = _bn_ss(_stats([ypre], NF), NF, gf, bef)
    gw = 16
    wgs = _ftf(ypre, cfp, ssf, kwp, pw2, gw)
    aggs = _sc_scatter(wgs, ft3, gw)
    ssv = _bn_ss(_gstats(aggs, nf2, gw), NV, gv, bev)
    return _gnorm(aggs, ssv, nf2, gw)


def kernel(inputs, face, full_nf_count, full_vt_map, filt_coeff, params):
    p = params
    face32 = face.astype(jnp.int32)
    ft3 = jnp.pad(face32.T, ((0, 0), (0, FP - NF))).reshape(3, 1, FP)
    cfp = jnp.pad(filt_coeff, ((0, FP - NF), (0, 128 - KS)))
    nf2 = full_nf_count.reshape(NV, 1)

    def wstack(i, cin_parts):
        dw = p['c%d_dw' % i]
        pw = p['c%d_v2f_pw' % i]
        full = dw[:, :, None] * pw[None, :, :]
        outs = []
        ofs = 0
        for cp in cin_parts:
            outs.append(full[:, ofs:ofs + cp, :])
            ofs += cp
        return outs

    def kwpad(i):
        return jnp.pad(p['c%d_kw' % i], ((0, 128 - KS), (0, 0)))

    def largs(i):
        return (kwpad(i), p['c%d_f2v_pw' % i],
                p['c%d_v2f_g' % i], p['c%d_v2f_be' % i],
                p['c%d_f2v_g' % i], p['c%d_f2v_be' % i],
                ft3, cfp, nf2)

    x0 = inputs
    neta = _layer([x0], wstack(0, [128]), *largs(0))
    netb = _layer([neta], wstack(1, [128]), *largs(1))
    netc = _layer([x0, netb], wstack(2, [128, 32]), *largs(2))
    netd = _layer([netc], wstack(3, [128]), *largs(3))

    tw = p['t_w']
    t = _mm([x0, netb, netd],
            [tw[None, :128], tw[None, 128:160], tw[None, 160:]], 1)[0]
    sst = _bn_ss(_stats([t], NV), NV, p['t_g'], p['t_be'])
    return _norm([t], sst)
